# baseline (speedup 1.0000x reference)
import functools

import jax
import jax.numpy as jnp
from jax import lax
from jax.experimental import pallas as pl
from jax.experimental.pallas import tpu as pltpu
from jax.experimental.pallas import tpu_sc as plsc

N_NODES = 40962
N_OUT = 10242
K = 7
N_ROWS = 1024
NUM_WORKERS = 32
SLAB = 8
NODES_PER_W = N_OUT // NUM_WORKERS
CHUNKS_PER_W = NODES_PER_W // 2
IDX_REGION = 168
N_CHUNKS = N_OUT // 2
IDX_ROWS = ((N_CHUNKS + 7) // 8) * 8
X_ROWS = N_NODES * SLAB
O_ROWS = N_OUT * SLAB
SCALE = 1.0 / K


@functools.partial(
    pl.kernel,
    mesh=plsc.VectorSubcoreMesh(core_axis_name="c", subcore_axis_name="s"),
    compiler_params=pltpu.CompilerParams(needs_layout_passes=False),
    out_type=jax.ShapeDtypeStruct((O_ROWS, 128), jnp.float32),
    scratch_types=[
        pltpu.VMEM((IDX_REGION * 128,), jnp.int32),
        pltpu.VMEM((128, 128), jnp.float32),
        pltpu.VMEM((128, 128), jnp.float32),
        pltpu.VMEM((16, 128), jnp.float32),
        pltpu.VMEM((16, 128), jnp.float32),
        pltpu.SemaphoreType.DMA,
        pltpu.SemaphoreType.DMA,
        pltpu.SemaphoreType.DMA,
        pltpu.SemaphoreType.DMA,
    ],
)
def _pool(x_hbm, idx_hbm, out_hbm, idx_v, gb0, gb1, st0, st1,
          sem_g0, sem_g1, sem_o0, sem_o1):
    wid = lax.axis_index("s") * 2 + lax.axis_index("c")
    pltpu.sync_copy(idx_hbm.at[pl.ds(wid * 20480, 20480)],
                    idx_v.at[pl.ds(0, 20480)])
    scale = jnp.float32(SCALE)
    gbufs = (gb0, gb1)
    gsems = (sem_g0, sem_g1)
    stages = (st0, st1)
    osems = (sem_o0, sem_o1)
    obase = wid * NODES_PER_W * SLAB

    def gather(j, p):
        return pltpu.async_copy(
            x_hbm.at[idx_v.at[pl.ds(j * 128, 128)]], gbufs[p], gsems[p])

    def compute(gb, st):
        def rr_step(rr, c2):
            for v in range(2):
                for m in range(8):
                    acc = gb[56 * v + rr, pl.ds(16 * m, 16)]
                    for k in range(1, K):
                        acc = acc + gb[56 * v + 8 * k + rr, pl.ds(16 * m, 16)]
                    st[8 * v + rr, pl.ds(16 * m, 16)] = acc * scale
            return c2
        lax.fori_loop(0, SLAB, rr_step, 0)

    gather(0, 0)
    gather(1, 1)

    def body(i, carry):
        for p in range(2):
            j = 2 * i + p
            pltpu.make_async_copy(
                x_hbm.at[idx_v.at[pl.ds(j * 128, 128)]], gbufs[p], gsems[p]).wait()

            @pl.when(i > 0)
            def _():
                pltpu.make_async_copy(
                    stages[p], out_hbm.at[pl.ds(obase, 16)], osems[p]).wait()

            compute(gbufs[p], stages[p])

            @pl.when(i < CHUNKS_PER_W // 2 - 1)
            def _():
                gather(j + 2, p)

            pltpu.async_copy(
                stages[p], out_hbm.at[pl.ds(obase + 16 * j, 16)], osems[p])
        return carry

    lax.fori_loop(0, CHUNKS_PER_W // 2, body, 0)
    pltpu.make_async_copy(
        stages[0], out_hbm.at[pl.ds(obase, 16)], sem_o0).wait()
    pltpu.make_async_copy(
        stages[1], out_hbm.at[pl.ds(obase, 16)], sem_o1).wait()

    @pl.when(wid == NUM_WORKERS - 1)
    def _():
        pltpu.sync_copy(idx_hbm.at[pl.ds(5120 * 128, 1024)],
                        idx_v.at[pl.ds(20480, 1024)])
        pltpu.async_copy(
            x_hbm.at[idx_v.at[pl.ds(20480, 128)]], gb0, sem_g0).wait()
        compute(gb0, st0)
        pltpu.sync_copy(st0, out_hbm.at[pl.ds(N_OUT * SLAB - 16, 16)])


def kernel(x, neigh_orders):
    B, D, N = x.shape
    ne = neigh_orders[: N_OUT * K].astype(jnp.int32).reshape(N_OUT, K)
    e = (ne[:, :, None] * SLAB + jnp.arange(SLAB, dtype=jnp.int32))
    e = e.reshape(N_CHUNKS, 112)
    e = jnp.pad(e, ((0, IDX_ROWS - N_CHUNKS), (0, 16))).reshape(-1)
    xa = x.reshape(B, 2, 128, N).transpose(3, 1, 0, 2).reshape(X_ROWS, 128)
    out = _pool(xa, e)
    out = out.reshape(N_OUT, 2, B, 128).transpose(2, 1, 3, 0)
    return out.reshape(B, D, N_OUT)


# speedup vs baseline: 4.5433x; 4.5433x over previous
import functools

import jax
import jax.numpy as jnp
from jax import lax
from jax.experimental import pallas as pl
from jax.experimental.pallas import tpu as pltpu
from jax.experimental.pallas import tpu_sc as plsc

N_NODES = 40962
N_OUT = 10242
K = 7
N_ROWS = 1024
NUM_WORKERS = 32
ROWS_PER_W = N_ROWS // NUM_WORKERS
ROW_TILES = (((N_NODES + 127) // 128 + 7) // 8) * 8
ROW_PAD = ROW_TILES * 128
OUT_TILES = (((N_OUT + 127) // 128 + 7) // 8) * 8
OUT_STRIDE = OUT_TILES * 128
VAL_TILES = (N_OUT + 127) // 128
QP = (VAL_TILES + 1) // 2
PK_ROWS = ((QP * K + 7) // 8) * 8
CHUNKS = ((0, 8, 16), (8, 16, 16), (16, 24, 16), (24, 32, 16),
          (32, 40, 16), (40, 41, 8))


@functools.partial(
    pl.kernel,
    mesh=plsc.VectorSubcoreMesh(core_axis_name="c", subcore_axis_name="s"),
    compiler_params=pltpu.CompilerParams(needs_layout_passes=False),
    out_type=jax.ShapeDtypeStruct((N_ROWS * OUT_TILES, 128), jnp.float32),
    scratch_types=[
        pltpu.VMEM((PK_ROWS, 128), jnp.float32),
        pltpu.VMEM((ROW_TILES, 128), jnp.float32),
        pltpu.VMEM((ROW_TILES, 128), jnp.float32),
        pltpu.VMEM((16, 128), jnp.float32),
        pltpu.VMEM((16, 128), jnp.float32),
        pltpu.SemaphoreType.DMA,
        pltpu.SemaphoreType.DMA,
        pltpu.SemaphoreType.DMA,
        pltpu.SemaphoreType.DMA,
    ],
)
def _pool(x_hbm, idx_hbm, out_hbm, idx_v, row_a, row_b, ob0, ob1,
          sem_a, sem_b, sem_o0, sem_o1):
    wid = lax.axis_index("s") * 2 + lax.axis_index("c")
    pltpu.sync_copy(idx_hbm, idx_v)
    scale = jnp.float32(1.0 / K)
    m16 = jnp.uint32(0xFFFF)
    m7 = jnp.uint32(127)
    obufs = (ob0, ob1)
    osems = (sem_o0, sem_o1)

    def process(row_v, row):
        pending = [None] * len(CHUNKS)
        for c, (qp0, qp1, nt) in enumerate(CHUNKS):
            buf = obufs[c % 2]
            if c >= 2:
                pending[c - 2].wait()

            def tile_qp(qp, c2, _qp0=qp0, _buf=buf):
                brow = 2 * (qp - _qp0)
                for e in range(2):
                    acc = [None] * 8
                    for k in range(K):
                        for h in range(4):
                            off = 448 * e + 64 * k + 16 * h
                            vecf = idx_v[7 * qp + off // 128,
                                         pl.ds(off % 128, 16)]
                            w = plsc.bitcast(vecf, jnp.uint32)
                            a = w & m16
                            b = w >> 16
                            va = plsc.load_gather(
                                row_v,
                                [plsc.bitcast(a >> 7, jnp.int32),
                                 plsc.bitcast(a & m7, jnp.int32)],
                            )
                            vb = plsc.load_gather(
                                row_v,
                                [plsc.bitcast(b >> 7, jnp.int32),
                                 plsc.bitcast(b & m7, jnp.int32)],
                            )
                            ia, ib = 2 * h, 2 * h + 1
                            if k == 0:
                                acc[ia], acc[ib] = va, vb
                            else:
                                acc[ia] = acc[ia] + va
                                acc[ib] = acc[ib] + vb
                    for h in range(4):
                        _buf[brow + e, pl.ds(32 * h, 16)] = (
                            acc[2 * h] * scale)
                        _buf[brow + e, pl.ds(32 * h + 16, 16)] = (
                            acc[2 * h + 1] * scale)
                return c2

            lax.fori_loop(qp0, qp1, tile_qp, 0)
            pending[c] = pltpu.async_copy(
                buf.at[pl.ds(0, nt)],
                out_hbm.at[pl.ds(row * OUT_TILES + 2 * qp0, nt)],
                osems[c % 2],
            )
        pending[len(CHUNKS) - 2].wait()
        pending[len(CHUNKS) - 1].wait()

    def pair_step(i, carry):
        r0 = wid * ROWS_PER_W + 2 * i
        h0 = pltpu.async_copy(
            x_hbm.at[pl.ds(r0 * ROW_TILES, ROW_TILES)], row_a, sem_a)
        h1 = pltpu.async_copy(
            x_hbm.at[pl.ds((r0 + 1) * ROW_TILES, ROW_TILES)], row_b, sem_b)
        h0.wait()
        process(row_a, r0)
        h1.wait()
        process(row_b, r0 + 1)
        return carry

    lax.fori_loop(0, ROWS_PER_W // 2, pair_step, 0)


def _pack_indices(neigh_orders):
    idx = neigh_orders[: N_OUT * K].astype(jnp.int32).reshape(N_OUT, K).T
    idx = jnp.pad(idx, ((0, 0), (0, QP * 256 - N_OUT)))
    a = idx.reshape(K, QP, 2, 4, 2, 16)
    packed = a[..., 0, :] | (a[..., 1, :] << 16)
    packed = packed.transpose(1, 2, 0, 3, 4).reshape(QP * K, 128)
    packed = jnp.pad(packed, ((0, PK_ROWS - QP * K), (0, 0)))
    return lax.bitcast_convert_type(packed, jnp.float32)


def kernel(x, neigh_orders):
    B, D, N = x.shape
    idx = _pack_indices(neigh_orders)
    xp = jnp.pad(x.reshape(B * D, N), ((0, 0), (0, ROW_PAD - N)))
    xp = xp.reshape(N_ROWS * ROW_TILES, 128)
    out = _pool(xp, idx)
    out = out.reshape(N_ROWS, OUT_STRIDE)[:, :N_OUT]
    return out.reshape(B, D, N_OUT)
